# baseline (speedup 1.0000x reference)
import functools

import jax
import jax.numpy as jnp
from jax import lax
from jax.experimental import pallas as pl
from jax.experimental.pallas import tpu as pltpu
from jax.experimental.pallas import tpu_sc as plsc

NC = 2
NS = 16
NW = NC * NS
EB = 128
C0_FRAC = 0.55


def _split(e):
    ksum = 4 * (-(-e // (NS * EB * 4)))
    k0 = 4 * int(round(C0_FRAC * ksum / 4))
    k0 = min(max(k0, 4), ksum - 4)
    return k0, ksum - k0


def _tile_range(c, s, k0, k1):
    is0 = c == 0
    nb = jnp.where(is0, k0, k1)
    base = jnp.where(is0, s * k0, NS * k0 + s * k1)
    return base, nb


def _sc_deg(k0, k1, n_pad):
    rows_per_tile = n_pad // NS
    kmax = max(k0, k1)
    mesh = plsc.VectorSubcoreMesh(core_axis_name="c", subcore_axis_name="s")

    def body(ones_hbm, dst_hbm, zeros_hbm, out_hbm, dst_v, ones_v, acc_sh):
        c = lax.axis_index("c")
        s = lax.axis_index("s")
        base, nb = _tile_range(c, s, k0, k1)
        r0 = pl.multiple_of(s * rows_per_tile, 8)
        pltpu.sync_copy(zeros_hbm.at[pl.ds(r0, rows_per_tile)],
                        acc_sh.at[pl.ds(r0, rows_per_tile)])
        pltpu.sync_copy(ones_hbm, ones_v)
        pltpu.sync_copy(dst_hbm.at[pl.ds(base, kmax)], dst_v)
        plsc.subcore_barrier()

        def step(j, carry):
            pltpu.sync_copy(ones_v, acc_sh.at[dst_v.at[j]], add=True)
            return carry

        lax.fori_loop(0, nb, step, 0)
        plsc.subcore_barrier()
        pltpu.sync_copy(acc_sh.at[pl.ds(r0, rows_per_tile)],
                        out_hbm.at[c, pl.ds(r0, rows_per_tile), pl.ds(0, 8)])

    return pl.kernel(
        body,
        out_type=jax.ShapeDtypeStruct((NC, n_pad, 128), jnp.float32),
        mesh=mesh,
        compiler_params=pltpu.CompilerParams(use_tc_tiling_on_sc=False),
        scratch_types=[
            pltpu.VMEM((kmax, EB), jnp.int32),
            pltpu.VMEM((EB, 8), jnp.float32),
            pltpu.VMEM_SHARED((n_pad, 8), jnp.float32),
        ],
    )


def _sc_agg(k0, k1, F, n_pad):
    rows_per_tile = n_pad // NS
    kmax = max(k0, k1)
    mesh = plsc.VectorSubcoreMesh(core_axis_name="c", subcore_axis_name="s")

    def body(p_hbm, src_hbm, dst_hbm, zeros_hbm, out_hbm,
             src_v, dst_v, rows0_v, rows1_v, rows2_v, rows3_v, acc_sh,
             sg0, sg1, sg2, sg3, ss0, ss1, ss2, ss3):
        c = lax.axis_index("c")
        s = lax.axis_index("s")
        base, nb = _tile_range(c, s, k0, k1)
        r0 = pl.multiple_of(s * rows_per_tile, 8)
        pltpu.sync_copy(zeros_hbm.at[pl.ds(r0, rows_per_tile)],
                        acc_sh.at[pl.ds(r0, rows_per_tile)])
        pltpu.sync_copy(src_hbm.at[pl.ds(base, kmax)], src_v)
        pltpu.sync_copy(dst_hbm.at[pl.ds(base, kmax)], dst_v)
        plsc.subcore_barrier()

        rows = (rows0_v, rows1_v, rows2_v, rows3_v)
        sgs = (sg0, sg1, sg2, sg3)
        sss = (ss0, ss1, ss2, ss3)

        def gath(j, slot):
            return pltpu.make_async_copy(p_hbm.at[src_v.at[j]], rows[slot],
                                         sgs[slot])

        def scat(j, slot):
            return pltpu.async_copy(rows[slot], acc_sh.at[dst_v.at[j]],
                                    sss[slot], add=True)

        def scat_wait(j, slot):
            pltpu.make_async_copy(rows[slot], acc_sh.at[dst_v.at[j]],
                                  sss[slot]).wait()

        for i in range(4):
            gath(i, i).start()

        def step(g, carry):
            j = g * 4
            for i in range(4):
                gath(j + i, i).wait()
                scat(j + i, i)
            for i in range(4):
                @pl.when(j + 4 + i < nb)
                def _(i=i):
                    scat_wait(j + i, i)
                    gath(j + 4 + i, i).start()
            return carry

        lax.fori_loop(0, nb // 4, step, 0)
        for i in range(4):
            scat_wait(nb - 4 + i, i)
        plsc.subcore_barrier()
        pltpu.sync_copy(acc_sh.at[pl.ds(r0, rows_per_tile)],
                        out_hbm.at[c, pl.ds(r0, rows_per_tile), pl.ds(0, F)])

    return pl.kernel(
        body,
        out_type=jax.ShapeDtypeStruct((NC, n_pad, 128), jnp.float32),
        mesh=mesh,
        compiler_params=pltpu.CompilerParams(use_tc_tiling_on_sc=False),
        scratch_types=[
            pltpu.VMEM((kmax, EB), jnp.int32),
            pltpu.VMEM((kmax, EB), jnp.int32),
            pltpu.VMEM((EB, F), jnp.float32),
            pltpu.VMEM((EB, F), jnp.float32),
            pltpu.VMEM((EB, F), jnp.float32),
            pltpu.VMEM((EB, F), jnp.float32),
            pltpu.VMEM_SHARED((n_pad, F), jnp.float32),
            pltpu.SemaphoreType.DMA,
            pltpu.SemaphoreType.DMA,
            pltpu.SemaphoreType.DMA,
            pltpu.SemaphoreType.DMA,
            pltpu.SemaphoreType.DMA,
            pltpu.SemaphoreType.DMA,
            pltpu.SemaphoreType.DMA,
            pltpu.SemaphoreType.DMA,
        ],
    )


def _tc_first(n, n_pad):
    def body(deg_ref, x_ref, w_ref, dinv_ref, p_ref):
        deg = deg_ref[0, :, 0:8] + deg_ref[1, :, 0:8] + 1.0
        dinv = lax.rsqrt(deg)
        dinv_ref[...] = dinv
        h = jnp.dot(x_ref[...], w_ref[...], preferred_element_type=jnp.float32)
        p_ref[...] = h * dinv[:n, 0:1]

    return pl.pallas_call(
        body,
        out_shape=(jax.ShapeDtypeStruct((n_pad, 8), jnp.float32),
                   jax.ShapeDtypeStruct((n, 16), jnp.float32)),
    )


def _tc_mid1(n, n_pad):
    def body(a_ref, p_ref, dinv_ref, b_ref, out_ref):
        scat = a_ref[0, :n, 0:16] + a_ref[1, :n, 0:16]
        dinv = dinv_ref[:n, 0:1]
        xn = jnp.maximum(dinv * (scat + p_ref[...]) + b_ref[...], 0.0)
        out_ref[...] = xn * dinv

    return pl.pallas_call(
        body,
        out_shape=jax.ShapeDtypeStruct((n, 16), jnp.float32),
    )


def _tc_mid2(n, n_pad):
    def body(a_ref, p_ref, dinv_ref, w2_ref, b2_ref, w3_ref, out_ref):
        scat = a_ref[0, :n, 0:16] + a_ref[1, :n, 0:16]
        dinv = dinv_ref[:n, 0:1]
        sx2 = dinv * (scat + p_ref[...])
        x3 = jnp.maximum(
            jnp.dot(sx2, w2_ref[...], preferred_element_type=jnp.float32)
            + b2_ref[...], 0.0)
        h3 = jnp.dot(x3, w3_ref[...], preferred_element_type=jnp.float32)
        out_ref[...] = h3 * dinv

    return pl.pallas_call(
        body,
        out_shape=jax.ShapeDtypeStruct((n, 16), jnp.float32),
    )


def _tc_last(n, n_pad):
    def body(a_ref, p_ref, dinv_ref, b_ref, wfc_ref, bfc_ref, out_ref):
        scat = a_ref[0, :n, 0:16] + a_ref[1, :n, 0:16]
        dinv = dinv_ref[:n, 0:1]
        xn = jnp.maximum(dinv * (scat + p_ref[...]) + b_ref[...], 0.0)
        ssum = jnp.sum(xn, axis=0, keepdims=True)
        out_ref[...] = (jnp.dot(ssum, wfc_ref[...],
                                preferred_element_type=jnp.float32) / n
                        + bfc_ref[...])

    return pl.pallas_call(
        body,
        out_shape=jax.ShapeDtypeStruct((1, 1), jnp.float32),
    )


def kernel(x, edge_index, W1, b1, W2, b2, W3, b3, Wfc, bfc):
    n = x.shape[0]
    e = edge_index.shape[1]
    n_pad = -(-(n + 1) // 128) * 128
    k0, k1 = _split(e)
    n_trash = n_pad - n
    b_alloc = NS * (k0 + k1) + (max(k0, k1) - min(k0, k1))
    e_pad = b_alloc * EB

    ei = edge_index.astype(jnp.int32)
    pad_i = jnp.arange(e_pad - e, dtype=jnp.int32)
    src = jnp.concatenate([ei[0], jnp.zeros((e_pad - e,), jnp.int32)])
    dst = jnp.concatenate([ei[1], n + pad_i % n_trash])
    src = src.reshape(b_alloc, EB)
    dst = dst.reshape(b_alloc, EB)

    ones8 = jnp.ones((EB, 8), jnp.float32)
    z8 = jnp.zeros((n_pad, 8), jnp.float32)
    z16 = jnp.zeros((n_pad, 16), jnp.float32)

    degparts = _sc_deg(k0, k1, n_pad)(ones8, dst, z8)
    dinv, p1 = _tc_first(n, n_pad)(degparts, x, W1)

    a1 = _sc_agg(k0, k1, 16, n_pad)(p1, src, dst, z16)
    p2 = _tc_mid1(n, n_pad)(a1, p1, dinv, b1.reshape(1, 16))

    a2 = _sc_agg(k0, k1, 16, n_pad)(p2, src, dst, z16)
    p3 = _tc_mid2(n, n_pad)(a2, p2, dinv, W2, b2.reshape(1, 32), W3)

    a3 = _sc_agg(k0, k1, 16, n_pad)(p3, src, dst, z16)
    out = _tc_last(n, n_pad)(a3, p3, dinv, b3.reshape(1, 16),
                             Wfc, bfc.reshape(1, 1))
    return out[0, 0]


# speedup vs baseline: 1.1004x; 1.1004x over previous
import functools

import jax
import jax.numpy as jnp
from jax import lax
from jax.experimental import pallas as pl
from jax.experimental.pallas import tpu as pltpu
from jax.experimental.pallas import tpu_sc as plsc

NC = 2
NS = 16
NW = NC * NS
EB = 128
C0_FRAC = 0.55


def _split(e):
    ksum = -(-e // (NS * EB))
    ksum += ksum % 2
    k0 = 2 * int(round(C0_FRAC * ksum / 2))
    k0 = min(max(k0, 2), ksum - 2)
    return k0, ksum - k0


def _tile_range(c, s, k0, k1):
    is0 = c == 0
    nb = jnp.where(is0, k0, k1)
    base = jnp.where(is0, s * k0, NS * k0 + s * k1)
    return base, nb


def _sc_deg(k0, k1, n_pad):
    rows_per_tile = n_pad // NS
    kmax = max(k0, k1)
    mesh = plsc.VectorSubcoreMesh(core_axis_name="c", subcore_axis_name="s")

    def body(ones_hbm, ei_hbm, zeros_hbm, out_hbm, dst_v, ones_v, acc_sh):
        c = lax.axis_index("c")
        s = lax.axis_index("s")
        base, nb = _tile_range(c, s, k0, k1)
        r0 = pl.multiple_of(s * rows_per_tile, 8)
        pltpu.sync_copy(zeros_hbm.at[pl.ds(r0, rows_per_tile)],
                        acc_sh.at[pl.ds(r0, rows_per_tile)])
        pltpu.sync_copy(ones_hbm, ones_v)
        pltpu.sync_copy(ei_hbm.at[1, pl.ds(base, kmax)], dst_v)
        plsc.subcore_barrier()

        def step(j, carry):
            pltpu.sync_copy(ones_v, acc_sh.at[dst_v.at[j]], add=True)
            return carry

        lax.fori_loop(0, nb, step, 0)
        plsc.subcore_barrier()
        pltpu.sync_copy(acc_sh.at[pl.ds(r0, rows_per_tile)],
                        out_hbm.at[c, pl.ds(r0, rows_per_tile), pl.ds(0, 8)])

    return pl.kernel(
        body,
        out_type=jax.ShapeDtypeStruct((NC, n_pad, 128), jnp.float32),
        mesh=mesh,
        compiler_params=pltpu.CompilerParams(use_tc_tiling_on_sc=False),
        scratch_types=[
            pltpu.VMEM((kmax, EB), jnp.int32),
            pltpu.VMEM((EB, 8), jnp.float32),
            pltpu.VMEM_SHARED((n_pad, 8), jnp.float32),
        ],
    )


def _sc_agg(k0, k1, F, n_pad):
    rows_per_tile = n_pad // NS
    kmax = max(k0, k1)
    mesh = plsc.VectorSubcoreMesh(core_axis_name="c", subcore_axis_name="s")

    def body(p_hbm, ei_hbm, zeros_hbm, out_hbm,
             src_v, dst_v, rows0_v, rows1_v, acc_sh,
             sg0, sg1, ss0, ss1):
        c = lax.axis_index("c")
        s = lax.axis_index("s")
        base, nb = _tile_range(c, s, k0, k1)
        r0 = pl.multiple_of(s * rows_per_tile, 8)
        pltpu.sync_copy(zeros_hbm.at[pl.ds(r0, rows_per_tile)],
                        acc_sh.at[pl.ds(r0, rows_per_tile)])
        pltpu.sync_copy(ei_hbm.at[0, pl.ds(base, kmax)], src_v)
        pltpu.sync_copy(ei_hbm.at[1, pl.ds(base, kmax)], dst_v)
        plsc.subcore_barrier()

        rows = (rows0_v, rows1_v)
        sgs = (sg0, sg1)
        sss = (ss0, ss1)

        def gath(j, slot):
            return pltpu.make_async_copy(p_hbm.at[src_v.at[j]], rows[slot],
                                         sgs[slot])

        def scat(j, slot):
            return pltpu.async_copy(rows[slot], acc_sh.at[dst_v.at[j]],
                                    sss[slot], add=True)

        def scat_wait(j, slot):
            pltpu.make_async_copy(rows[slot], acc_sh.at[dst_v.at[j]],
                                  sss[slot]).wait()

        gath(0, 0).start()
        gath(1, 1).start()

        def step(g, carry):
            j = g * 2
            gath(j, 0).wait()
            scat(j, 0)
            gath(j + 1, 1).wait()
            scat(j + 1, 1)

            @pl.when(j + 2 < nb)
            def _():
                scat_wait(j, 0)
                gath(j + 2, 0).start()

            @pl.when(j + 3 < nb)
            def _():
                scat_wait(j + 1, 1)
                gath(j + 3, 1).start()

            return carry

        lax.fori_loop(0, nb // 2, step, 0)
        scat_wait(nb - 2, 0)
        scat_wait(nb - 1, 1)
        plsc.subcore_barrier()
        pltpu.sync_copy(acc_sh.at[pl.ds(r0, rows_per_tile)],
                        out_hbm.at[c, pl.ds(r0, rows_per_tile), pl.ds(0, F)])

    return pl.kernel(
        body,
        out_type=jax.ShapeDtypeStruct((NC, n_pad, 128), jnp.float32),
        mesh=mesh,
        compiler_params=pltpu.CompilerParams(use_tc_tiling_on_sc=False),
        scratch_types=[
            pltpu.VMEM((kmax, EB), jnp.int32),
            pltpu.VMEM((kmax, EB), jnp.int32),
            pltpu.VMEM((EB, F), jnp.float32),
            pltpu.VMEM((EB, F), jnp.float32),
            pltpu.VMEM_SHARED((n_pad, F), jnp.float32),
            pltpu.SemaphoreType.DMA,
            pltpu.SemaphoreType.DMA,
            pltpu.SemaphoreType.DMA,
            pltpu.SemaphoreType.DMA,
        ],
    )


def _tc_first(n, n_pad):
    def body(deg_ref, x_ref, w_ref, dinv_ref, p_ref):
        deg = deg_ref[0, :, 0:8] + deg_ref[1, :, 0:8] + 1.0
        dinv = lax.rsqrt(deg)
        dinv_ref[...] = dinv
        h = jnp.dot(x_ref[...], w_ref[...], preferred_element_type=jnp.float32)
        p_ref[...] = h * dinv[:n, 0:1]

    return pl.pallas_call(
        body,
        out_shape=(jax.ShapeDtypeStruct((n_pad, 8), jnp.float32),
                   jax.ShapeDtypeStruct((n, 16), jnp.float32)),
    )


def _tc_mid1(n, n_pad):
    def body(a_ref, p_ref, dinv_ref, b_ref, out_ref):
        scat = a_ref[0, :n, 0:16] + a_ref[1, :n, 0:16]
        dinv = dinv_ref[:n, 0:1]
        xn = jnp.maximum(dinv * (scat + p_ref[...]) + b_ref[...], 0.0)
        out_ref[...] = xn * dinv

    return pl.pallas_call(
        body,
        out_shape=jax.ShapeDtypeStruct((n, 16), jnp.float32),
    )


def _tc_mid2(n, n_pad):
    def body(a_ref, p_ref, dinv_ref, w2_ref, b2_ref, w3_ref, out_ref):
        scat = a_ref[0, :n, 0:16] + a_ref[1, :n, 0:16]
        dinv = dinv_ref[:n, 0:1]
        sx2 = dinv * (scat + p_ref[...])
        x3 = jnp.maximum(
            jnp.dot(sx2, w2_ref[...], preferred_element_type=jnp.float32)
            + b2_ref[...], 0.0)
        h3 = jnp.dot(x3, w3_ref[...], preferred_element_type=jnp.float32)
        out_ref[...] = h3 * dinv

    return pl.pallas_call(
        body,
        out_shape=jax.ShapeDtypeStruct((n, 16), jnp.float32),
    )


def _tc_last(n, n_pad):
    def body(a_ref, p_ref, dinv_ref, b_ref, wfc_ref, bfc_ref, out_ref):
        scat = a_ref[0, :n, 0:16] + a_ref[1, :n, 0:16]
        dinv = dinv_ref[:n, 0:1]
        xn = jnp.maximum(dinv * (scat + p_ref[...]) + b_ref[...], 0.0)
        ssum = jnp.sum(xn, axis=0, keepdims=True)
        out_ref[...] = (jnp.dot(ssum, wfc_ref[...],
                                preferred_element_type=jnp.float32) / n
                        + bfc_ref[...])

    return pl.pallas_call(
        body,
        out_shape=jax.ShapeDtypeStruct((1, 1), jnp.float32),
    )


def kernel(x, edge_index, W1, b1, W2, b2, W3, b3, Wfc, bfc):
    n = x.shape[0]
    e = edge_index.shape[1]
    n_pad = -(-(n + 1) // 128) * 128
    k0, k1 = _split(e)
    n_trash = n_pad - n
    b_alloc = NS * (k0 + k1) + (max(k0, k1) - min(k0, k1))
    e_pad = b_alloc * EB

    ei = edge_index.astype(jnp.int32)
    pad_i = jnp.arange(e_pad - e, dtype=jnp.int32)
    pad_block = jnp.stack([jnp.zeros((e_pad - e,), jnp.int32),
                           n + pad_i % n_trash])
    ei3 = jnp.concatenate([ei, pad_block], axis=1).reshape(2, b_alloc, EB)

    ones8 = jnp.ones((EB, 8), jnp.float32)
    z8 = jnp.zeros((n_pad, 8), jnp.float32)
    z16 = jnp.zeros((n_pad, 16), jnp.float32)

    degparts = _sc_deg(k0, k1, n_pad)(ones8, ei3, z8)
    dinv, p1 = _tc_first(n, n_pad)(degparts, x, W1)

    a1 = _sc_agg(k0, k1, 16, n_pad)(p1, ei3, z16)
    p2 = _tc_mid1(n, n_pad)(a1, p1, dinv, b1.reshape(1, 16))

    a2 = _sc_agg(k0, k1, 16, n_pad)(p2, ei3, z16)
    p3 = _tc_mid2(n, n_pad)(a2, p2, dinv, W2, b2.reshape(1, 32), W3)

    a3 = _sc_agg(k0, k1, 16, n_pad)(p3, ei3, z16)
    out = _tc_last(n, n_pad)(a3, p3, dinv, b3.reshape(1, 16),
                             Wfc, bfc.reshape(1, 1))
    return out[0, 0]
